# Initial kernel scaffold; baseline (speedup 1.0000x reference)
import jax
import jax.numpy as jnp
from jax.experimental import pallas as pl


def kernel(state, enc_W1, enc_b1, enc_W2, enc_b2, enc_W3, enc_b3, dec_W1, dec_b1, dec_W2, dec_b2, dec_W3, dec_b3, codebooks):
    raise NotImplementedError("write your pallas kernel here")


# speedup vs baseline: 1.6603x; 1.6603x over previous
import jax
import jax.numpy as jnp
from jax.experimental import pallas as pl

B_BLK = 512
G = 4
K = 512


def _dot(a, b):
    return jax.lax.dot_general(
        a, b, (((1,), (0,)), ((), ())),
        precision=jax.lax.Precision.HIGHEST,
        preferred_element_type=jnp.float32)


def _dot_fast(a, b):
    return jax.lax.dot_general(
        a.astype(jnp.bfloat16), b.astype(jnp.bfloat16),
        (((1,), (0,)), ((), ())),
        preferred_element_type=jnp.float32)


def _vqvae_body(x_ref, ew1, eb1, ew2, eb2, ew3, eb3,
                dw1, db1, dw2, db2, dw3, db3, cb_ref, cbt_ref,
                code_ref, loss_ref):
    i = pl.program_id(0)
    x = x_ref[...]
    h = jnp.maximum(_dot_fast(x, ew1[...]) + eb1[...], 0.0)
    h = jnp.maximum(_dot_fast(h, ew2[...]) + eb2[...], 0.0)
    z = _dot_fast(h, ew3[...]) + eb3[...]

    blk = x.shape[0]
    lane_iota = jax.lax.broadcasted_iota(jnp.int32, (blk, K), 1)
    resid = z
    quant = jnp.zeros_like(z)
    dmin_sums = []
    for g in range(G):
        cb = cb_ref[g]
        cbt = cbt_ref[g]
        norms = jnp.sum(cbt * cbt, axis=0, keepdims=True)
        rn = jnp.sum(resid * resid, axis=1, keepdims=True)
        d = rn - 2.0 * _dot_fast(resid, cbt) + norms
        m = jnp.min(d, axis=1, keepdims=True)
        idx = jnp.min(jnp.where(d == m, lane_iota, K), axis=1).astype(jnp.int32)
        onehot = (lane_iota == idx[:, None]).astype(jnp.float32)
        q = _dot(onehot, cb)
        quant = quant + q
        resid = resid - q
        code_ref[g, :] = idx
        dmin_sums.append(jnp.sum(m))

    y = jnp.maximum(_dot_fast(quant, dw1[...]) + db1[...], 0.0)
    y = jnp.maximum(_dot_fast(y, dw2[...]) + db2[...], 0.0)
    dec = _dot_fast(y, dw3[...]) + db3[...]
    diff = x - dec
    rows = [jnp.sum(jnp.abs(diff)), jnp.sum(diff * diff)] + dmin_sums
    part = jnp.concatenate(
        [jnp.full((1, 128), r, jnp.float32) for r in rows], axis=0)

    @pl.when(i == 0)
    def _():
        loss_ref[...] = part

    @pl.when(i != 0)
    def _():
        loss_ref[...] += part


def kernel(state, enc_W1, enc_b1, enc_W2, enc_b2, enc_W3, enc_b3,
           dec_W1, dec_b1, dec_W2, dec_b2, dec_W3, dec_b3, codebooks):
    b = state.shape[0]
    x = state.reshape(b, -1)
    d_in = x.shape[1]
    h = enc_W1.shape[1]
    d = enc_W3.shape[1]
    cbt = codebooks.transpose(0, 2, 1)

    grid = b // B_BLK
    full = lambda shp: pl.BlockSpec(shp, lambda i, _s=None: tuple(0 for _ in shp))
    codes, losses = pl.pallas_call(
        _vqvae_body,
        grid=(grid,),
        in_specs=[
            pl.BlockSpec((B_BLK, d_in), lambda i: (i, 0)),
            full((d_in, h)), full((1, h)),
            full((h, h)), full((1, h)),
            full((h, d)), full((1, d)),
            full((d, h)), full((1, h)),
            full((h, h)), full((1, h)),
            full((h, d_in)), full((1, d_in)),
            full((G, K, d)), full((G, d, K)),
        ],
        out_specs=[
            pl.BlockSpec((G, B_BLK), lambda i: (0, i)),
            pl.BlockSpec((6, 128), lambda i: (0, 0)),
        ],
        out_shape=[
            jax.ShapeDtypeStruct((G, b), jnp.int32),
            jax.ShapeDtypeStruct((6, 128), jnp.float32),
        ],
    )(x, enc_W1, enc_b1.reshape(1, h), enc_W2, enc_b2.reshape(1, h),
      enc_W3, enc_b3.reshape(1, d), dec_W1, dec_b1.reshape(1, h),
      dec_W2, dec_b2.reshape(1, h), dec_W3, dec_b3.reshape(1, d_in),
      codebooks, cbt)

    sums = losses[:, 0]
    encoder_loss = sums[0] / (b * d_in)
    vqvae_recon_loss = sums[1] / (b * d_in)
    vq_loss_sum = jnp.sum(sums[2:2 + G]) / (b * d)
    loss = encoder_loss * 1.0 + vq_loss_sum * 5.0
    vq_code = codes.T
    return (loss, vq_code, vq_loss_sum, vqvae_recon_loss, encoder_loss)


# speedup vs baseline: 2.4204x; 1.4578x over previous
import jax
import jax.numpy as jnp
from jax.experimental import pallas as pl

B_BLK = 512
G = 4
K = 512


def _dot(a, b):
    return jax.lax.dot_general(
        a, b, (((1,), (0,)), ((), ())),
        precision=jax.lax.Precision.HIGHEST,
        preferred_element_type=jnp.float32)


def _dot_fast(a, b):
    return jax.lax.dot_general(
        a.astype(jnp.bfloat16), b.astype(jnp.bfloat16),
        (((1,), (0,)), ((), ())),
        preferred_element_type=jnp.float32)


def _select(onehot, hi, lo):
    return _dot_fast(onehot, hi) + _dot_fast(onehot, lo)


def _vqvae_body(x_ref, ew1, eb1, ew2, eb2, ew3, eb3,
                dw1, db1, dw2, db2, dw3, db3, cbh_ref, cbl_ref, cbt_ref,
                code_ref, loss_ref):
    i = pl.program_id(0)
    x = x_ref[...]
    h = jnp.maximum(_dot_fast(x, ew1[...]) + eb1[...], 0.0)
    h = jnp.maximum(_dot_fast(h, ew2[...]) + eb2[...], 0.0)
    z = _dot_fast(h, ew3[...]) + eb3[...]

    blk = x.shape[0]
    lane_iota = jax.lax.broadcasted_iota(jnp.int32, (blk, K), 1)
    resid = z
    quant = jnp.zeros_like(z)
    dmin_sums = []
    for g in range(G):
        cbt = cbt_ref[g]
        norms = jnp.sum(cbt * cbt, axis=0, keepdims=True)
        rn = jnp.sum(resid * resid, axis=1, keepdims=True)
        d = rn - 2.0 * _dot_fast(resid, cbt) + norms
        m = jnp.min(d, axis=1, keepdims=True)
        idx = jnp.min(jnp.where(d == m, lane_iota, K), axis=1).astype(jnp.int32)
        onehot = (lane_iota == idx[:, None]).astype(jnp.bfloat16)
        q = _select(onehot, cbh_ref[g], cbl_ref[g])
        quant = quant + q
        resid = resid - q
        code_ref[g, :] = idx
        dmin_sums.append(jnp.sum(m))

    y = jnp.maximum(_dot_fast(quant, dw1[...]) + db1[...], 0.0)
    y = jnp.maximum(_dot_fast(y, dw2[...]) + db2[...], 0.0)
    dec = _dot_fast(y, dw3[...]) + db3[...]
    diff = x - dec
    rows = [jnp.sum(jnp.abs(diff)), jnp.sum(diff * diff)] + dmin_sums
    part = jnp.concatenate(
        [jnp.full((1, 128), r, jnp.float32) for r in rows], axis=0)

    @pl.when(i == 0)
    def _():
        loss_ref[...] = part

    @pl.when(i != 0)
    def _():
        loss_ref[...] += part


def kernel(state, enc_W1, enc_b1, enc_W2, enc_b2, enc_W3, enc_b3,
           dec_W1, dec_b1, dec_W2, dec_b2, dec_W3, dec_b3, codebooks):
    b = state.shape[0]
    x = state.reshape(b, -1)
    d_in = x.shape[1]
    h = enc_W1.shape[1]
    d = enc_W3.shape[1]
    cbt = codebooks.transpose(0, 2, 1)
    cb_hi = codebooks.astype(jnp.bfloat16)
    cb_lo = (codebooks - cb_hi.astype(jnp.float32)).astype(jnp.bfloat16)

    grid = b // B_BLK
    full = lambda shp: pl.BlockSpec(shp, lambda i, _s=None: tuple(0 for _ in shp))
    codes, losses = pl.pallas_call(
        _vqvae_body,
        grid=(grid,),
        in_specs=[
            pl.BlockSpec((B_BLK, d_in), lambda i: (i, 0)),
            full((d_in, h)), full((1, h)),
            full((h, h)), full((1, h)),
            full((h, d)), full((1, d)),
            full((d, h)), full((1, h)),
            full((h, h)), full((1, h)),
            full((h, d_in)), full((1, d_in)),
            full((G, K, d)), full((G, K, d)), full((G, d, K)),
        ],
        out_specs=[
            pl.BlockSpec((G, B_BLK), lambda i: (0, i)),
            pl.BlockSpec((6, 128), lambda i: (0, 0)),
        ],
        out_shape=[
            jax.ShapeDtypeStruct((G, b), jnp.int32),
            jax.ShapeDtypeStruct((6, 128), jnp.float32),
        ],
    )(x, enc_W1, enc_b1.reshape(1, h), enc_W2, enc_b2.reshape(1, h),
      enc_W3, enc_b3.reshape(1, d), dec_W1, dec_b1.reshape(1, h),
      dec_W2, dec_b2.reshape(1, h), dec_W3, dec_b3.reshape(1, d_in),
      cb_hi, cb_lo, cbt)

    sums = losses[:, 0]
    encoder_loss = sums[0] / (b * d_in)
    vqvae_recon_loss = sums[1] / (b * d_in)
    vq_loss_sum = jnp.sum(sums[2:2 + G]) / (b * d)
    loss = encoder_loss * 1.0 + vq_loss_sum * 5.0
    vq_code = codes.T
    return (loss, vq_code, vq_loss_sum, vqvae_recon_loss, encoder_loss)
